# baseline (speedup 1.0000x reference)
import jax
import jax.numpy as jnp
from jax.experimental import pallas as pl

_BS = 512


def _add_kernel(x_ref, w_ref, o_ref):
    o_ref[0] = x_ref[0] + w_ref[...]


def kernel(x, weight):
    B, S, H = x.shape
    w = weight[:S]
    grid = (S // _BS, B)
    return pl.pallas_call(
        _add_kernel,
        grid=grid,
        in_specs=[
            pl.BlockSpec((1, _BS, H), lambda i, j: (j, i, 0)),
            pl.BlockSpec((_BS, H), lambda i, j: (i, 0)),
        ],
        out_specs=pl.BlockSpec((1, _BS, H), lambda i, j: (j, i, 0)),
        out_shape=jax.ShapeDtypeStruct(x.shape, x.dtype),
    )(x, w)


# speedup vs baseline: 1.1621x; 1.1621x over previous
import jax
import jax.numpy as jnp
from jax.experimental import pallas as pl

_BS = 2048


def _add_kernel(x_ref, w_ref, o_ref):
    o_ref[0] = x_ref[0] + w_ref[...]


def kernel(x, weight):
    B, S, H = x.shape
    w = weight[:S]
    grid = (S // _BS, B)
    return pl.pallas_call(
        _add_kernel,
        grid=grid,
        in_specs=[
            pl.BlockSpec((1, _BS, H), lambda i, j: (j, i, 0)),
            pl.BlockSpec((_BS, H), lambda i, j: (i, 0)),
        ],
        out_specs=pl.BlockSpec((1, _BS, H), lambda i, j: (j, i, 0)),
        out_shape=jax.ShapeDtypeStruct(x.shape, x.dtype),
    )(x, w)
